# baseline (speedup 1.0000x reference)
import functools

import numpy as np
import jax
import jax.numpy as jnp
from jax.experimental import pallas as pl
from jax.experimental.pallas import tpu as pltpu


def _elu(x):
    return jnp.where(x > 0.0, x, jnp.exp(jnp.minimum(x, 0.0)) - 1.0)


def _bias_col(b_ref, n, B):
    col = jnp.concatenate(
        [jnp.full((1, 1), b_ref[c], jnp.float32) for c in range(n)], axis=0)
    return jnp.concatenate([col] * B, axis=0)


def _layer(X, w_ref, m_ref, up_ref, bias, y_scr, B, elu):
    S, C_out, C_in = w_ref.shape
    V_in = X.shape[1]
    V_out = up_ref.shape[2]
    mask = m_ref[...]
    for s in range(S):
        w = w_ref[s]
        wrow = jnp.concatenate([w] * B, axis=1)
        wt = jnp.concatenate([wrow] * B, axis=0)
        Wb = wt * mask
        y_scr[:, s * V_in:(s + 1) * V_in] = jnp.dot(
            Wb, X, preferred_element_type=jnp.float32)
    Up = up_ref[...].reshape(S * V_in, V_out)
    O = jnp.dot(y_scr[...], Up, preferred_element_type=jnp.float32) + bias
    return _elu(O) if elu else O


def _decoder_kernel(z0_ref, z1_ref, wp_ref, bp_ref,
                    w0_ref, m0_ref, up0_ref,
                    w1_ref, m1_ref, up1_ref,
                    w2_ref, m2_ref, up2_ref,
                    b0_ref, b1_ref, b2_ref,
                    o_ref, y0_scr, y1_scr, y2_scr, *, B, C0, V0):
    C1 = w0_ref.shape[1]
    C2 = w1_ref.shape[1]
    C3 = w2_ref.shape[1]
    b0 = _bias_col(b0_ref, C1, B)
    b1 = _bias_col(b1_ref, C2, B)
    b2 = jnp.concatenate(
        [jnp.full((B, 1), b2_ref[c], jnp.float32) for c in range(C3)], axis=0)

    @pl.when(pl.program_id(1) == 0)
    def _front():
        z = jnp.concatenate([z0_ref[...], z1_ref[...]], axis=1)
        y = jnp.dot(z, wp_ref[...],
                    preferred_element_type=jnp.float32) + bp_ref[...]
        X = y.reshape(B * C0, V0)
        X = _layer(X, w0_ref, m0_ref, up0_ref, b0, y0_scr, B, elu=True)
        X = _layer(X, w1_ref, m1_ref, up1_ref, b1, y1_scr, B, elu=True)
        S, _, _ = w2_ref.shape
        V_in = X.shape[1]
        mask = m2_ref[...]
        for s in range(S):
            w = w2_ref[s]
            wrep = jnp.repeat(w, B, axis=0)
            wt = jnp.concatenate([wrep] * B, axis=1)
            Wb = wt * mask
            y2_scr[:, s * V_in:(s + 1) * V_in] = jnp.dot(
                Wb, X, preferred_element_type=jnp.float32)

    V_out = up2_ref.shape[2]
    K = y2_scr.shape[1]
    Up = up2_ref[...].reshape(K, V_out)
    O = jnp.dot(y2_scr[...], Up, preferred_element_type=jnp.float32) + b2
    o_ref[...] = O.reshape(C3, B, V_out)


def kernel(z0, z1, proj_fused_w, proj_fused_b,
           upT_0, wT_0, b_col_0,
           upT_1, wT_1, b_col_1,
           upT_2, wT_2, b_col_2):
    B = z0.shape[0]
    S, V0, V1 = upT_0.shape
    V2 = upT_1.shape[2]
    V3 = upT_2.shape[2]
    Z = proj_fused_w.shape[0]
    C0 = proj_fused_w.shape[1] // V0
    C1, C2, C3 = wT_0.shape[1], wT_1.shape[1], wT_2.shape[1]
    f32 = jnp.float32

    def blk_mask(co, ci):
        r = np.arange(B * co)[:, None] // co
        c = np.arange(B * ci)[None, :] // ci
        return jnp.asarray(r == c, dtype=f32)

    def blk_mask_cmajor(co, ci):
        r = np.arange(co * B)[:, None] % B
        c = np.arange(B * ci)[None, :] // ci
        return jnp.asarray(r == c, dtype=f32)

    m0, m1 = blk_mask(C1, C0), blk_mask(C2, C1)
    m2 = blk_mask_cmajor(C3, C2)

    NC = 2
    T = 2
    Vc = V3 // (NC * T)

    out = pl.pallas_call(
        functools.partial(_decoder_kernel, B=B, C0=C0, V0=V0),
        out_shape=jax.ShapeDtypeStruct((C3, B, V3), f32),
        grid=(NC, T),
        in_specs=[
            pl.BlockSpec((B, z0.shape[1]), lambda i, t: (0, 0)),
            pl.BlockSpec((B, z1.shape[1]), lambda i, t: (0, 0)),
            pl.BlockSpec((Z, C0 * V0), lambda i, t: (0, 0)),
            pl.BlockSpec((1, C0 * V0), lambda i, t: (0, 0)),
            pl.BlockSpec((S, C1, C0), lambda i, t: (0, 0, 0)),
            pl.BlockSpec((B * C1, B * C0), lambda i, t: (0, 0)),
            pl.BlockSpec((S, V0, V1), lambda i, t: (0, 0, 0)),
            pl.BlockSpec((S, C2, C1), lambda i, t: (0, 0, 0)),
            pl.BlockSpec((B * C2, B * C1), lambda i, t: (0, 0)),
            pl.BlockSpec((S, V1, V2), lambda i, t: (0, 0, 0)),
            pl.BlockSpec((S, C3, C2), lambda i, t: (0, 0, 0)),
            pl.BlockSpec((C3 * B, B * C2), lambda i, t: (0, 0)),
            pl.BlockSpec((S, V2, Vc), lambda i, t: (0, 0, i * T + t)),
            pl.BlockSpec(memory_space=pltpu.SMEM),
            pl.BlockSpec(memory_space=pltpu.SMEM),
            pl.BlockSpec(memory_space=pltpu.SMEM),
        ],
        out_specs=pl.BlockSpec((C3, B, Vc), lambda i, t: (0, 0, i * T + t)),
        scratch_shapes=[
            pltpu.VMEM((B * C1, S * V0), f32),
            pltpu.VMEM((B * C2, S * V1), f32),
            pltpu.VMEM((C3 * B, S * V2), f32),
        ],
        compiler_params=pltpu.CompilerParams(
            dimension_semantics=("parallel", "arbitrary"),
        ),
    )(z0, z1, proj_fused_w, proj_fused_b,
      wT_0, m0, upT_0,
      wT_1, m1, upT_1,
      wT_2, m2, upT_2,
      b_col_0.reshape(C1), b_col_1.reshape(C2), b_col_2.reshape(C3))

    return jnp.transpose(out, (1, 2, 0))


# speedup vs baseline: 1.1093x; 1.1093x over previous
import functools

import numpy as np
import jax
import jax.numpy as jnp
from jax.experimental import pallas as pl
from jax.experimental.pallas import tpu as pltpu


def _elu(x):
    return jnp.where(x > 0.0, x, jnp.exp(jnp.minimum(x, 0.0)) - 1.0)


def _bias_col(b_ref, n, B):
    col = jnp.concatenate(
        [jnp.full((1, 1), b_ref[c], jnp.float32) for c in range(n)], axis=0)
    return jnp.concatenate([col] * B, axis=0)


def _layer(X, w_ref, m_ref, up_ref, bias, y_scr, B, elu):
    S, C_out, C_in = w_ref.shape
    V_in = X.shape[1]
    V_out = up_ref.shape[2]
    mask = m_ref[...]
    for s in range(S):
        w = w_ref[s]
        wrow = jnp.concatenate([w] * B, axis=1)
        wt = jnp.concatenate([wrow] * B, axis=0)
        Wb = wt * mask
        y_scr[:, s * V_in:(s + 1) * V_in] = jnp.dot(
            Wb, X, preferred_element_type=jnp.float32)
    Up = up_ref[...].reshape(S * V_in, V_out)
    O = jnp.dot(y_scr[...], Up, preferred_element_type=jnp.float32) + bias
    return _elu(O) if elu else O


def _decoder_kernel(z0_ref, z1_ref, wp_ref, bp_ref,
                    w0_ref, m0_ref, up0_ref,
                    w1_ref, m1_ref, up1_ref,
                    w2_ref, m2_ref, up2_ref,
                    b0_ref, b1_ref, b2_ref,
                    o_ref, y0_scr, y1_scr, y2_scr, *, B, C0, V0):
    C1 = w0_ref.shape[1]
    C2 = w1_ref.shape[1]
    C3 = w2_ref.shape[1]
    b0 = _bias_col(b0_ref, C1, B)
    b1 = _bias_col(b1_ref, C2, B)
    b2 = jnp.concatenate(
        [jnp.full((B, 1), b2_ref[c], jnp.float32) for c in range(C3)], axis=0)
    z = jnp.concatenate([z0_ref[...], z1_ref[...]], axis=1)
    y = jnp.dot(z, wp_ref[...], preferred_element_type=jnp.float32) + bp_ref[...]
    X = y.reshape(B * C0, V0)
    X = _layer(X, w0_ref, m0_ref, up0_ref, b0, y0_scr, B, elu=True)
    X = _layer(X, w1_ref, m1_ref, up1_ref, b1, y1_scr, B, elu=True)
    S, _, _ = w2_ref.shape
    V_in = X.shape[1]
    V_out = up2_ref.shape[2]
    mask = m2_ref[...]
    for s in range(S):
        w = w2_ref[s]
        wrep = jnp.repeat(w, B, axis=0)
        wt = jnp.concatenate([wrep] * B, axis=1)
        Wb = wt * mask
        y2_scr[:, s * V_in:(s + 1) * V_in] = jnp.dot(
            Wb, X, preferred_element_type=jnp.float32)
    Up = up2_ref[...].reshape(S * V_in, V_out)
    O = jnp.dot(y2_scr[...], Up, preferred_element_type=jnp.float32) + b2
    o_ref[...] = O.reshape(C3, B, V_out)


def kernel(z0, z1, proj_fused_w, proj_fused_b,
           upT_0, wT_0, b_col_0,
           upT_1, wT_1, b_col_1,
           upT_2, wT_2, b_col_2):
    B = z0.shape[0]
    S, V0, V1 = upT_0.shape
    V2 = upT_1.shape[2]
    V3 = upT_2.shape[2]
    Z = proj_fused_w.shape[0]
    C0 = proj_fused_w.shape[1] // V0
    C1, C2, C3 = wT_0.shape[1], wT_1.shape[1], wT_2.shape[1]
    f32 = jnp.float32

    def blk_mask(co, ci):
        r = np.arange(B * co)[:, None] // co
        c = np.arange(B * ci)[None, :] // ci
        return jnp.asarray(r == c, dtype=f32)

    def blk_mask_cmajor(co, ci):
        r = np.arange(co * B)[:, None] % B
        c = np.arange(B * ci)[None, :] // ci
        return jnp.asarray(r == c, dtype=f32)

    m0, m1 = blk_mask(C1, C0), blk_mask(C2, C1)
    m2 = blk_mask_cmajor(C3, C2)

    NC = 2
    Vc = V3 // NC

    out = pl.pallas_call(
        functools.partial(_decoder_kernel, B=B, C0=C0, V0=V0),
        out_shape=jax.ShapeDtypeStruct((C3, B, V3), f32),
        grid=(NC,),
        in_specs=[
            pl.BlockSpec((B, z0.shape[1]), lambda i: (0, 0)),
            pl.BlockSpec((B, z1.shape[1]), lambda i: (0, 0)),
            pl.BlockSpec((Z, C0 * V0), lambda i: (0, 0)),
            pl.BlockSpec((1, C0 * V0), lambda i: (0, 0)),
            pl.BlockSpec((S, C1, C0), lambda i: (0, 0, 0)),
            pl.BlockSpec((B * C1, B * C0), lambda i: (0, 0)),
            pl.BlockSpec((S, V0, V1), lambda i: (0, 0, 0)),
            pl.BlockSpec((S, C2, C1), lambda i: (0, 0, 0)),
            pl.BlockSpec((B * C2, B * C1), lambda i: (0, 0)),
            pl.BlockSpec((S, V1, V2), lambda i: (0, 0, 0)),
            pl.BlockSpec((S, C3, C2), lambda i: (0, 0, 0)),
            pl.BlockSpec((C3 * B, B * C2), lambda i: (0, 0)),
            pl.BlockSpec((S, V2, Vc), lambda i: (0, 0, i)),
            pl.BlockSpec(memory_space=pltpu.SMEM),
            pl.BlockSpec(memory_space=pltpu.SMEM),
            pl.BlockSpec(memory_space=pltpu.SMEM),
        ],
        out_specs=pl.BlockSpec((C3, B, Vc), lambda i: (0, 0, i)),
        scratch_shapes=[
            pltpu.VMEM((B * C1, S * V0), f32),
            pltpu.VMEM((B * C2, S * V1), f32),
            pltpu.VMEM((C3 * B, S * V2), f32),
        ],
        compiler_params=pltpu.CompilerParams(
            dimension_semantics=("parallel",),
        ),
    )(z0, z1, proj_fused_w, proj_fused_b,
      wT_0, m0, upT_0,
      wT_1, m1, upT_1,
      wT_2, m2, upT_2,
      b_col_0.reshape(C1), b_col_1.reshape(C2), b_col_2.reshape(C3))

    return jnp.transpose(out, (1, 2, 0))
